# baseline (speedup 1.0000x reference)
import functools

import jax
import jax.numpy as jnp
from jax import lax
from jax.experimental import pallas as pl
from jax.experimental.pallas import tpu as pltpu
from jax.experimental.pallas import tpu_sc as plsc

NBATCH = 4096
NP = 50
D = 256
NPAIR = NP // 2
NC, NS = 2, 16
NW = NC * NS
ROWS_PER_W = NBATCH // NW
GRP = 4
N_GRP = ROWS_PER_W // GRP
NBUF = 3
DEPTH = 2
REP = 128

_mesh = plsc.VectorSubcoreMesh(core_axis_name="c", subcore_axis_name="s")


def _pidx_tc_body(lab_ref, out_ref):
    lab = lab_ref[...].astype(jnp.float32)
    j = lax.broadcasted_iota(jnp.int32, (NP, NPAIR), 0)
    jj = lax.broadcasted_iota(jnp.int32, (NP, NPAIR), 1)
    m = jnp.where(j == 2 * jj, 2.0, 0.0) + jnp.where(j == 2 * jj + 1, 1.0, 0.0)
    base = jnp.dot(lab, m.astype(jnp.float32),
                   preferred_element_type=jnp.float32).astype(jnp.int32)
    r = lax.broadcasted_iota(jnp.int32, (NBATCH, NPAIR), 0)
    c = lax.broadcasted_iota(jnp.int32, (NBATCH, NPAIR), 1)
    salt = jnp.bitwise_and(r * NPAIR + c, REP - 1)
    out_ref[...] = base + 4 * salt


_pidx_tc = pl.pallas_call(
    _pidx_tc_body,
    out_shape=jax.ShapeDtypeStruct((NBATCH, NPAIR), jnp.int32),
)


@functools.partial(
    pl.kernel,
    mesh=_mesh,
    out_type=jax.ShapeDtypeStruct((NBATCH, NP, D), jnp.float16),
    scratch_types=[
        pltpu.VMEM((ROWS_PER_W, NPAIR), jnp.int32),
        pltpu.VMEM((NBUF, GRP, NPAIR, D), jnp.int32),
        pltpu.SemaphoreType.DMA((NBUF,)),
    ],
)
def _embed_lookup(ptable_hbm, pidx_hbm, out_hbm, pidx_v, rows_v, sem_g):
    wid = lax.axis_index("s") * NC + lax.axis_index("c")
    row0 = wid * ROWS_PER_W
    out_i32 = out_hbm.bitcast(jnp.int32)

    pltpu.sync_copy(pidx_hbm.at[pl.ds(row0, ROWS_PER_W)], pidx_v)

    def gather_descs(g):
        b = lax.rem(g, NBUF)
        return [
            pltpu.make_async_copy(
                ptable_hbm.at[pidx_v.at[g * GRP + k]],
                rows_v.at[b, k],
                sem_g.at[b],
            )
            for k in range(GRP)
        ]

    for g in range(DEPTH):
        for dsc in gather_descs(g):
            dsc.start()

    def body(g, carry):
        b = lax.rem(g, NBUF)
        for dsc in gather_descs(g):
            dsc.wait()
        @pl.when(g + DEPTH < N_GRP)
        def _():
            for dsc in gather_descs(g + DEPTH):
                dsc.start()
        pltpu.sync_copy(rows_v.at[b], out_i32.at[pl.ds(row0 + g * GRP, GRP)])
        return carry

    lax.fori_loop(0, N_GRP, body, 0)


def kernel(points, labels, point_embed):
    del points
    e = point_embed
    ptable_f16 = jnp.stack([
        jnp.stack([e[0], e[0]], axis=-1),
        jnp.stack([e[0], e[1]], axis=-1),
        jnp.stack([e[1], e[0]], axis=-1),
        jnp.stack([e[1], e[1]], axis=-1),
    ])
    ptable = jax.lax.bitcast_convert_type(ptable_f16, jnp.int32)
    ptable_rep = jnp.tile(ptable, (REP, 1))
    pidx = _pidx_tc(labels.astype(jnp.int32))
    return _embed_lookup(ptable_rep, pidx)


# speedup vs baseline: 1.0610x; 1.0610x over previous
import functools

import jax
import jax.numpy as jnp
from jax import lax
from jax.experimental import pallas as pl
from jax.experimental.pallas import tpu as pltpu
from jax.experimental.pallas import tpu_sc as plsc

NBATCH = 4096
NP = 50
D = 256
NPAIR = NP // 2
NC, NS = 2, 16
NW = NC * NS
ROWS_PER_W = NBATCH // NW
GRP = 4
N_GRP = ROWS_PER_W // GRP
NBUF = 3
DEPTH = 2
REP = 128

_mesh = plsc.VectorSubcoreMesh(core_axis_name="c", subcore_axis_name="s")


def _prep_tc_body(lab_ref, pe_ref, pidx_ref, pt_ref):
    lab = lab_ref[...].astype(jnp.float32)
    j = lax.broadcasted_iota(jnp.int32, (NP, NPAIR), 0)
    jj = lax.broadcasted_iota(jnp.int32, (NP, NPAIR), 1)
    m = jnp.where(j == 2 * jj, 2.0, 0.0) + jnp.where(j == 2 * jj + 1, 1.0, 0.0)
    base = jnp.dot(lab, m.astype(jnp.float32),
                   preferred_element_type=jnp.float32).astype(jnp.int32)
    r = lax.broadcasted_iota(jnp.int32, (NBATCH, NPAIR), 0)
    c = lax.broadcasted_iota(jnp.int32, (NBATCH, NPAIR), 1)
    salt = jnp.bitwise_and(r * NPAIR + c, REP - 1)
    pidx_ref[...] = base + 4 * salt

    e = pe_ref[...]
    p = lax.broadcasted_iota(jnp.int32, (4 * REP, D), 0)
    lo = jnp.where(jnp.bitwise_and(p, 2) == 0, e[0:1, :], e[1:2, :])
    hi = jnp.where(jnp.bitwise_and(p, 1) == 0, e[0:1, :], e[1:2, :])
    pt_ref[...] = lax.bitcast_convert_type(
        lo | (hi << jnp.uint32(16)), jnp.int32)


_prep_tc = pl.pallas_call(
    _prep_tc_body,
    out_shape=(
        jax.ShapeDtypeStruct((NBATCH, NPAIR), jnp.int32),
        jax.ShapeDtypeStruct((4 * REP, D), jnp.int32),
    ),
)


@functools.partial(
    pl.kernel,
    mesh=_mesh,
    out_type=jax.ShapeDtypeStruct((NBATCH, NP, D), jnp.float16),
    scratch_types=[
        pltpu.VMEM((ROWS_PER_W, NPAIR), jnp.int32),
        pltpu.VMEM((NBUF, GRP, NPAIR, D), jnp.int32),
        pltpu.SemaphoreType.DMA((NBUF,)),
    ],
)
def _embed_lookup(ptable_hbm, pidx_hbm, out_hbm, pidx_v, rows_v, sem_g):
    wid = lax.axis_index("s") * NC + lax.axis_index("c")
    row0 = wid * ROWS_PER_W
    out_i32 = out_hbm.bitcast(jnp.int32)

    pltpu.sync_copy(pidx_hbm.at[pl.ds(row0, ROWS_PER_W)], pidx_v)

    def gather_descs(g):
        b = lax.rem(g, NBUF)
        return [
            pltpu.make_async_copy(
                ptable_hbm.at[pidx_v.at[g * GRP + k]],
                rows_v.at[b, k],
                sem_g.at[b],
            )
            for k in range(GRP)
        ]

    for g in range(DEPTH):
        for dsc in gather_descs(g):
            dsc.start()

    def body(g, carry):
        b = lax.rem(g, NBUF)
        for dsc in gather_descs(g):
            dsc.wait()
        @pl.when(g + DEPTH < N_GRP)
        def _():
            for dsc in gather_descs(g + DEPTH):
                dsc.start()
        pltpu.sync_copy(rows_v.at[b], out_i32.at[pl.ds(row0 + g * GRP, GRP)])
        return carry

    lax.fori_loop(0, N_GRP, body, 0)


def kernel(points, labels, point_embed):
    del points
    pe_bits = jax.lax.bitcast_convert_type(
        point_embed, jnp.uint16).astype(jnp.uint32)
    pidx, ptable_rep = _prep_tc(labels.astype(jnp.int32), pe_bits)
    return _embed_lookup(ptable_rep, pidx)
